# baseline (speedup 1.0000x reference)
import functools

import jax
import jax.numpy as jnp
from jax import lax
from jax.experimental import pallas as pl
from jax.experimental.pallas import tpu as pltpu
from jax.experimental.pallas import tpu_sc as plsc

D = 128
NC, NS = 2, 16
NW = NC * NS
IDXW = 128


def _sc_gather(atab, ctab, ptab, aidx2d, cidx2d, pidx2d, dstidx, zeros, B, L):
    BPW = B // NW
    ROWS = BPW * L // IDXW
    PROWS = BPW // IDXW
    NPASS = 4
    HPW = BPW // NPASS
    ROWSP = ROWS // NPASS
    mesh = plsc.VectorSubcoreMesh(core_axis_name="c", subcore_axis_name="s")

    @functools.partial(
        pl.kernel,
        mesh=mesh,
        out_type=(
            jax.ShapeDtypeStruct((B, D), jnp.float32),
            jax.ShapeDtypeStruct((B, D), jnp.float32),
            jax.ShapeDtypeStruct((B, D), jnp.float32),
        ),
        scratch_types=[
            pltpu.VMEM((ROWS, IDXW), jnp.int32),
            pltpu.VMEM((ROWS, IDXW), jnp.int32),
            pltpu.VMEM((ROWS, IDXW), jnp.int32),
            pltpu.VMEM((PROWS, IDXW), jnp.int32),
            pltpu.VMEM((IDXW, D), jnp.float32),
            pltpu.VMEM((IDXW, D), jnp.float32),
            pltpu.VMEM((IDXW, D), jnp.float32),
            pltpu.VMEM((IDXW, D), jnp.float32),
            pltpu.VMEM((IDXW, D), jnp.float32),
            pltpu.VMEM_SHARED((NS * HPW, D), jnp.float32),
            pltpu.SemaphoreType.DMA,
            pltpu.SemaphoreType.DMA,
            pltpu.SemaphoreType.DMA,
            pltpu.SemaphoreType.DMA,
            pltpu.SemaphoreType.DMA,
            pltpu.SemaphoreType.DMA,
            pltpu.SemaphoreType.DMA,
            pltpu.SemaphoreType.DMA,
            pltpu.SemaphoreType.DMA,
            pltpu.SemaphoreType.DMA,
            pltpu.SemaphoreType.DMA,
        ],
    )
    def k(atab_h, ctab_h, ptab_h, aidx_h, cidx_h, pidx_h, dst_h, z_h,
          a_out, p_out, c_out, idx_va, idx_vc, dst_v, idx_vp,
          buf0, buf1, buf2, buf3, buf4,
          acc, g0, g1, g2, g3, g4, s0, s1, s2, s3, s4, msem):
        cix = lax.axis_index("c")
        six = lax.axis_index("s")
        wid = six * NC + cix
        base = wid * BPW
        bufs = (buf0, buf1, buf2, buf3, buf4)
        gsem = (g0, g1, g2, g3, g4)
        ssem = (s0, s1, s2, s3, s4)
        NBUF = 5
        accme = acc.at[pl.ds(six * HPW, HPW)]

        pre = [
            pltpu.async_copy(dst_h.at[six], dst_v, msem),
            pltpu.async_copy(aidx_h.at[pl.ds(wid * ROWS, ROWS)], idx_va, msem),
            pltpu.async_copy(cidx_h.at[pl.ds(wid * ROWS, ROWS)], idx_vc, msem),
            pltpu.async_copy(pidx_h.at[pl.ds(wid * PROWS, PROWS)], idx_vp, msem),
            pltpu.async_copy(z_h, accme, msem),
        ]
        for cp in pre:
            cp.wait()

        def run_pipe(n, start_gather, start_drain, bufset, nbuf):
            g = [None] * n
            s = [None] * n
            issued = 0
            for j in range(n):
                while issued < n and issued < j + nbuf:
                    bi = issued % nbuf
                    if issued >= nbuf:
                        s[issued - nbuf].wait()
                    g[issued] = start_gather(issued, bufset[bi], gsem[bi])
                    issued += 1
                g[j].wait()
                s[j] = start_drain(j, bufset[j % nbuf], ssem[j % nbuf])
            for j in range(max(0, n - nbuf), n):
                s[j].wait()

        for p in range(NPASS):
            for tab_h, idx_v, out in ((atab_h, idx_va, a_out),
                                      (ctab_h, idx_vc, c_out)):
                run_pipe(
                    ROWSP,
                    lambda j, b, sm, t=tab_h, iv=idx_v, r0=p * ROWSP:
                        pltpu.async_copy(t.at[iv.at[r0 + j]], b, sm),
                    lambda j, b, sm, r0=p * ROWSP: pltpu.async_copy(
                        b, acc.at[dst_v.at[r0 + j]], sm, add=True),
                    bufs, NBUF,
                )
                pltpu.sync_copy(accme, out.at[pl.ds(base + p * HPW, HPW)])
                if not (p == NPASS - 1 and tab_h is ctab_h):
                    pltpu.sync_copy(z_h, accme)
        run_pipe(
            PROWS,
            lambda j, b, sm: pltpu.async_copy(ptab_h.at[idx_vp.at[j]], b, sm),
            lambda j, b, sm: pltpu.async_copy(
                b, p_out.at[pl.ds(base + j * IDXW, IDXW)], sm),
            bufs, 2,
        )

    return k(atab, ctab, ptab, aidx2d, cidx2d, pidx2d, dstidx, zeros)


def _mlp_pre(desc, W1d, b1):
    B, DD = desc.shape
    L1 = W1d.shape[1]
    BM = 512

    def body(desc_ref, W_ref, b_ref, out_ref):
        x = desc_ref[...].astype(jnp.bfloat16)
        h = jnp.dot(x, W_ref[...], preferred_element_type=jnp.float32)
        out_ref[...] = (h + b_ref[...]).astype(jnp.bfloat16)

    return pl.pallas_call(
        body,
        grid=(B // BM,),
        in_specs=[
            pl.BlockSpec((BM, DD), lambda i: (i, 0)),
            pl.BlockSpec((DD, L1), lambda i: (0, 0)),
            pl.BlockSpec((1, L1), lambda i: (0, 0)),
        ],
        out_specs=pl.BlockSpec((BM, L1), lambda i: (i, 0)),
        out_shape=jax.ShapeDtypeStruct((B, L1), jnp.bfloat16),
    )(desc, W1d.astype(jnp.bfloat16), b1.reshape(1, -1))


def _mlp_post(h1, asum, prow, csum, aidx, cidx, W1e, W2, b2):
    B, L1 = h1.shape
    BM = 512
    L = aidx.shape[1]
    E = W1e.shape[0]
    L2 = W2.shape[1]

    def body(h1_ref, asum_ref, p_ref, csum_ref, aidx_ref, cidx_ref,
             W1e_ref, W2_ref, b2_ref, out_ref):
        acnt = jnp.maximum(jnp.sum((aidx_ref[...] != 0).astype(jnp.float32),
                                   axis=1, keepdims=True), 1.0)
        ccnt = jnp.maximum(jnp.sum((cidx_ref[...] != 0).astype(jnp.float32),
                                   axis=1, keepdims=True), 1.0)
        a = asum_ref[...] / acnt
        c = csum_ref[...] / ccnt
        e = jnp.concatenate([a, p_ref[...], c], axis=1).astype(jnp.bfloat16)
        h = h1_ref[...].astype(jnp.float32) + jnp.dot(
            e, W1e_ref[...], preferred_element_type=jnp.float32)
        h = jnp.maximum(h, 0.0).astype(jnp.bfloat16)
        out_ref[...] = jnp.dot(h, W2_ref[...],
                               preferred_element_type=jnp.float32) + b2_ref[...]

    return pl.pallas_call(
        body,
        grid=(B // BM,),
        in_specs=[
            pl.BlockSpec((BM, L1), lambda i: (i, 0)),
            pl.BlockSpec((BM, D), lambda i: (i, 0)),
            pl.BlockSpec((BM, D), lambda i: (i, 0)),
            pl.BlockSpec((BM, D), lambda i: (i, 0)),
            pl.BlockSpec((BM, L), lambda i: (i, 0)),
            pl.BlockSpec((BM, L), lambda i: (i, 0)),
            pl.BlockSpec((E, L1), lambda i: (0, 0)),
            pl.BlockSpec((L1, L2), lambda i: (0, 0)),
            pl.BlockSpec((1, L2), lambda i: (0, 0)),
        ],
        out_specs=pl.BlockSpec((BM, L2), lambda i: (i, 0)),
        out_shape=jax.ShapeDtypeStruct((B, L2), jnp.float32),
    )(h1, asum, prow, csum, aidx, cidx, W1e.astype(jnp.bfloat16),
      W2.astype(jnp.bfloat16), b2.reshape(1, -1))


def kernel(description_embedding, authors, publishers, categories,
           authors_table, publishers_table, categories_table,
           W1, b1, W2, b2):
    B, L = authors.shape
    aidx32 = authors.astype(jnp.int32)
    cidx32 = categories.astype(jnp.int32)
    aidx2d = aidx32.reshape(-1, IDXW)
    cidx2d = cidx32.reshape(-1, IDXW)
    pidx2d = publishers.astype(jnp.int32).reshape(-1, IDXW)
    BPW = B // NW
    ROWS = BPW * L // IDXW
    HPW = BPW // 4
    pat = ((jnp.arange(BPW * L, dtype=jnp.int32) // L) % HPW).reshape(ROWS, IDXW)
    dstidx = pat[None, :, :] + (jnp.arange(NS, dtype=jnp.int32) * HPW)[:, None, None]
    zeros = jnp.zeros((HPW, D), jnp.float32)
    asum, prow, csum = _sc_gather(authors_table, categories_table,
                                  publishers_table, aidx2d, cidx2d, pidx2d,
                                  dstidx, zeros, B, L)
    DD = description_embedding.shape[1]
    h1 = _mlp_pre(description_embedding, W1[:DD], b1)
    return _mlp_post(h1, asum, prow, csum, aidx32, cidx32, W1[DD:], W2, b2)


# speedup vs baseline: 1.0399x; 1.0399x over previous
import functools

import jax
import jax.numpy as jnp
from jax import lax
from jax.experimental import pallas as pl
from jax.experimental.pallas import tpu as pltpu
from jax.experimental.pallas import tpu_sc as plsc

D = 128
NC, NS = 2, 16
NW = NC * NS
IDXW = 128


def _sc_gather(atab, ctab, ptab, aidx2d, cidx2d, pidx2d, dstidx, zeros, B, L):
    BPW = B // NW
    ROWS = BPW * L // IDXW
    PROWS = BPW // IDXW
    NPASS = 2
    HPW = BPW // NPASS
    ROWSP = ROWS // NPASS
    mesh = plsc.VectorSubcoreMesh(core_axis_name="c", subcore_axis_name="s")

    @functools.partial(
        pl.kernel,
        mesh=mesh,
        out_type=(
            jax.ShapeDtypeStruct((B, D), jnp.float32),
            jax.ShapeDtypeStruct((B, D), jnp.float32),
            jax.ShapeDtypeStruct((B, D), jnp.float32),
        ),
        scratch_types=[
            pltpu.VMEM((ROWS, IDXW), jnp.int32),
            pltpu.VMEM((ROWS, IDXW), jnp.int32),
            pltpu.VMEM((ROWS, IDXW), jnp.int32),
            pltpu.VMEM((PROWS, IDXW), jnp.int32),
            pltpu.VMEM((IDXW, D), jnp.float32),
            pltpu.VMEM((IDXW, D), jnp.float32),
            pltpu.VMEM((IDXW, D), jnp.float32),
            pltpu.VMEM((IDXW, D), jnp.float32),
            pltpu.VMEM_SHARED((NS * HPW, D), jnp.float32),
            pltpu.SemaphoreType.DMA,
            pltpu.SemaphoreType.DMA,
            pltpu.SemaphoreType.DMA,
            pltpu.SemaphoreType.DMA,
            pltpu.SemaphoreType.DMA,
            pltpu.SemaphoreType.DMA,
            pltpu.SemaphoreType.DMA,
            pltpu.SemaphoreType.DMA,
            pltpu.SemaphoreType.DMA,
        ],
    )
    def k(atab_h, ctab_h, ptab_h, aidx_h, cidx_h, pidx_h, dst_h, z_h,
          a_out, p_out, c_out, idx_va, idx_vc, dst_v, idx_vp,
          buf0, buf1, buf2, buf3,
          acc, g0, g1, g2, g3, s0, s1, s2, s3, msem):
        cix = lax.axis_index("c")
        six = lax.axis_index("s")
        wid = six * NC + cix
        base = wid * BPW
        bufs = (buf0, buf1, buf2, buf3)
        gsem = (g0, g1, g2, g3)
        ssem = (s0, s1, s2, s3)
        NBUF = 4
        accme = acc.at[pl.ds(six * HPW, HPW)]

        pre = [
            pltpu.async_copy(dst_h.at[six], dst_v, msem),
            pltpu.async_copy(aidx_h.at[pl.ds(wid * ROWS, ROWS)], idx_va, msem),
            pltpu.async_copy(cidx_h.at[pl.ds(wid * ROWS, ROWS)], idx_vc, msem),
            pltpu.async_copy(pidx_h.at[pl.ds(wid * PROWS, PROWS)], idx_vp, msem),
            pltpu.async_copy(z_h, accme, msem),
        ]
        for cp in pre:
            cp.wait()

        chunk_g = []
        chunk_d = []
        seg_of = []
        segs = []
        for p in range(NPASS):
            for tab_h, idx_v, out in ((atab_h, idx_va, a_out),
                                      (ctab_h, idx_vc, c_out)):
                for jj in range(ROWSP):
                    r = p * ROWSP + jj
                    chunk_g.append(
                        lambda b, sm, t=tab_h, iv=idx_v, r=r:
                            pltpu.async_copy(t.at[iv.at[r]], b, sm))
                    chunk_d.append(
                        lambda b, sm, r=r: pltpu.async_copy(
                            b, acc.at[dst_v.at[r]], sm, add=True))
                    seg_of.append(len(segs))
                segs.append((out, p))
        for jj in range(PROWS):
            chunk_g.append(
                lambda b, sm, r=jj:
                    pltpu.async_copy(ptab_h.at[idx_vp.at[r]], b, sm))
            chunk_d.append(
                lambda b, sm, r=jj: pltpu.async_copy(
                    b, p_out.at[pl.ds(base + r * IDXW, IDXW)], sm))
            seg_of.append(len(segs))
        segs.append(None)

        n = len(chunk_g)
        g = [None] * n
        s = [None] * n
        s_waited = [False] * n

        def wait_s(i):
            if not s_waited[i]:
                s[i].wait()
                s_waited[i] = True

        issued = 0
        for j in range(n):
            while issued < n and issued < j + NBUF:
                bi = issued % NBUF
                if issued >= NBUF:
                    wait_s(issued - NBUF)
                g[issued] = chunk_g[issued](bufs[bi], gsem[bi])
                issued += 1
            if j > 0 and seg_of[j] != seg_of[j - 1]:
                for i in range(max(0, j - NBUF), j):
                    wait_s(i)
                out, p = segs[seg_of[j - 1]]
                pltpu.sync_copy(accme, out.at[pl.ds(base + p * HPW, HPW)])
                if segs[seg_of[j]] is not None:
                    pltpu.sync_copy(z_h, accme)
            g[j].wait()
            s[j] = chunk_d[j](bufs[j % NBUF], ssem[j % NBUF])
        for i in range(max(0, n - NBUF), n):
            wait_s(i)

    return k(atab, ctab, ptab, aidx2d, cidx2d, pidx2d, dstidx, zeros)


def _mlp_pre(desc, W1d, b1):
    B, DD = desc.shape
    L1 = W1d.shape[1]
    BM = 512

    def body(desc_ref, W_ref, b_ref, out_ref):
        x = desc_ref[...].astype(jnp.bfloat16)
        h = jnp.dot(x, W_ref[...], preferred_element_type=jnp.float32)
        out_ref[...] = (h + b_ref[...]).astype(jnp.bfloat16)

    return pl.pallas_call(
        body,
        grid=(B // BM,),
        in_specs=[
            pl.BlockSpec((BM, DD), lambda i: (i, 0)),
            pl.BlockSpec((DD, L1), lambda i: (0, 0)),
            pl.BlockSpec((1, L1), lambda i: (0, 0)),
        ],
        out_specs=pl.BlockSpec((BM, L1), lambda i: (i, 0)),
        out_shape=jax.ShapeDtypeStruct((B, L1), jnp.bfloat16),
    )(desc, W1d.astype(jnp.bfloat16), b1.reshape(1, -1))


def _mlp_post(h1, asum, prow, csum, aidx, cidx, W1e, W2, b2):
    B, L1 = h1.shape
    BM = 512
    L = aidx.shape[1]
    E = W1e.shape[0]
    L2 = W2.shape[1]

    def body(h1_ref, asum_ref, p_ref, csum_ref, aidx_ref, cidx_ref,
             W1e_ref, W2_ref, b2_ref, out_ref):
        acnt = jnp.maximum(jnp.sum((aidx_ref[...] != 0).astype(jnp.float32),
                                   axis=1, keepdims=True), 1.0)
        ccnt = jnp.maximum(jnp.sum((cidx_ref[...] != 0).astype(jnp.float32),
                                   axis=1, keepdims=True), 1.0)
        a = asum_ref[...] / acnt
        c = csum_ref[...] / ccnt
        e = jnp.concatenate([a, p_ref[...], c], axis=1).astype(jnp.bfloat16)
        h = h1_ref[...].astype(jnp.float32) + jnp.dot(
            e, W1e_ref[...], preferred_element_type=jnp.float32)
        h = jnp.maximum(h, 0.0).astype(jnp.bfloat16)
        out_ref[...] = jnp.dot(h, W2_ref[...],
                               preferred_element_type=jnp.float32) + b2_ref[...]

    return pl.pallas_call(
        body,
        grid=(B // BM,),
        in_specs=[
            pl.BlockSpec((BM, L1), lambda i: (i, 0)),
            pl.BlockSpec((BM, D), lambda i: (i, 0)),
            pl.BlockSpec((BM, D), lambda i: (i, 0)),
            pl.BlockSpec((BM, D), lambda i: (i, 0)),
            pl.BlockSpec((BM, L), lambda i: (i, 0)),
            pl.BlockSpec((BM, L), lambda i: (i, 0)),
            pl.BlockSpec((E, L1), lambda i: (0, 0)),
            pl.BlockSpec((L1, L2), lambda i: (0, 0)),
            pl.BlockSpec((1, L2), lambda i: (0, 0)),
        ],
        out_specs=pl.BlockSpec((BM, L2), lambda i: (i, 0)),
        out_shape=jax.ShapeDtypeStruct((B, L2), jnp.float32),
    )(h1, asum, prow, csum, aidx, cidx, W1e.astype(jnp.bfloat16),
      W2.astype(jnp.bfloat16), b2.reshape(1, -1))


def kernel(description_embedding, authors, publishers, categories,
           authors_table, publishers_table, categories_table,
           W1, b1, W2, b2):
    B, L = authors.shape
    aidx32 = authors.astype(jnp.int32)
    cidx32 = categories.astype(jnp.int32)
    aidx2d = aidx32.reshape(-1, IDXW)
    cidx2d = cidx32.reshape(-1, IDXW)
    pidx2d = publishers.astype(jnp.int32).reshape(-1, IDXW)
    BPW = B // NW
    ROWS = BPW * L // IDXW
    HPW = BPW // 2
    pat = ((jnp.arange(BPW * L, dtype=jnp.int32) // L) % HPW).reshape(ROWS, IDXW)
    dstidx = pat[None, :, :] + (jnp.arange(NS, dtype=jnp.int32) * HPW)[:, None, None]
    zeros = jnp.zeros((HPW, D), jnp.float32)
    asum, prow, csum = _sc_gather(authors_table, categories_table,
                                  publishers_table, aidx2d, cidx2d, pidx2d,
                                  dstidx, zeros, B, L)
    DD = description_embedding.shape[1]
    h1 = _mlp_pre(description_embedding, W1[:DD], b1)
    return _mlp_post(h1, asum, prow, csum, aidx32, cidx32, W1[DD:], W2, b2)


# speedup vs baseline: 1.0559x; 1.0154x over previous
import functools

import jax
import jax.numpy as jnp
from jax import lax
from jax.experimental import pallas as pl
from jax.experimental.pallas import tpu as pltpu
from jax.experimental.pallas import tpu_sc as plsc

D = 128
NC, NS = 2, 16
NW = NC * NS
IDXW = 128


def _sc_gather(atab, ctab, ptab, aidx2d, cidx2d, pidx2d, dstidx, zeros, B, L):
    BPW = B // NW
    ROWS = BPW * L // IDXW
    PROWS = BPW // IDXW
    NPASS = 2
    HPW = BPW // NPASS
    ROWSP = ROWS // NPASS
    mesh = plsc.VectorSubcoreMesh(core_axis_name="c", subcore_axis_name="s")

    @functools.partial(
        pl.kernel,
        mesh=mesh,
        out_type=(
            jax.ShapeDtypeStruct((B, D), jnp.float32),
            jax.ShapeDtypeStruct((B, D), jnp.float32),
            jax.ShapeDtypeStruct((B, D), jnp.float32),
        ),
        scratch_types=[
            pltpu.VMEM((ROWS, IDXW), jnp.int32),
            pltpu.VMEM((ROWS, IDXW), jnp.int32),
            pltpu.VMEM((ROWS, IDXW), jnp.int32),
            pltpu.VMEM((PROWS, IDXW), jnp.int32),
            pltpu.VMEM((IDXW, D), jnp.float32),
            pltpu.VMEM((IDXW, D), jnp.float32),
            pltpu.VMEM((IDXW, D), jnp.float32),
            pltpu.VMEM((IDXW, D), jnp.float32),
            pltpu.VMEM_SHARED((NS * HPW, D), jnp.float32),
            pltpu.SemaphoreType.DMA,
            pltpu.SemaphoreType.DMA,
            pltpu.SemaphoreType.DMA,
            pltpu.SemaphoreType.DMA,
            pltpu.SemaphoreType.DMA,
            pltpu.SemaphoreType.DMA,
            pltpu.SemaphoreType.DMA,
            pltpu.SemaphoreType.DMA,
            pltpu.SemaphoreType.DMA,
        ],
    )
    def k(atab_h, ctab_h, ptab_h, aidx_h, cidx_h, pidx_h, dst_h, z_h,
          a_out, p_out, c_out, idx_va, idx_vc, dst_v, idx_vp,
          buf0, buf1, buf2, buf3,
          acc, g0, g1, g2, g3, s0, s1, s2, s3, msem):
        cix = lax.axis_index("c")
        six = lax.axis_index("s")
        wid = six * NC + cix
        base = wid * BPW
        bufs = (buf0, buf1, buf2, buf3)
        gsem = (g0, g1, g2, g3)
        ssem = (s0, s1, s2, s3)
        NBUF = 4
        accme = acc.at[pl.ds(six * HPW, HPW)]

        pre = [
            pltpu.async_copy(dst_h.at[six], dst_v, msem),
            pltpu.async_copy(aidx_h.at[pl.ds(wid * ROWS, ROWS)], idx_va, msem),
            pltpu.async_copy(cidx_h.at[pl.ds(wid * ROWS, ROWS)], idx_vc, msem),
            pltpu.async_copy(pidx_h.at[pl.ds(wid * PROWS, PROWS)], idx_vp, msem),
            pltpu.async_copy(z_h, accme, msem),
        ]
        for cp in pre:
            cp.wait()

        chunk_g = []
        chunk_d = []
        seg_of = []
        segs = []
        for p in range(NPASS):
            for tab_h, idx_v, out in ((atab_h, idx_va, a_out),
                                      (ctab_h, idx_vc, c_out)):
                for jj in range(ROWSP):
                    r = p * ROWSP + jj
                    chunk_g.append(
                        lambda b, sm, t=tab_h, iv=idx_v, r=r:
                            pltpu.async_copy(t.at[iv.at[r]], b, sm))
                    chunk_d.append(
                        lambda b, sm, r=r: pltpu.async_copy(
                            b, acc.at[dst_v.at[r]], sm, add=True))
                    seg_of.append(len(segs))
                segs.append((out, p))
        for jj in range(PROWS):
            chunk_g.append(
                lambda b, sm, r=jj:
                    pltpu.async_copy(ptab_h.at[idx_vp.at[r]], b, sm))
            chunk_d.append(
                lambda b, sm, r=jj: pltpu.async_copy(
                    b, p_out.at[pl.ds(base + r * IDXW, IDXW)], sm))
            seg_of.append(len(segs))
        segs.append(None)

        n = len(chunk_g)
        g = [None] * n
        s = [None] * n
        s_waited = [False] * n

        def wait_s(i):
            if not s_waited[i]:
                s[i].wait()
                s_waited[i] = True

        issued = 0
        for j in range(n):
            while issued < n and issued < j + NBUF:
                bi = issued % NBUF
                if issued >= NBUF:
                    wait_s(issued - NBUF)
                g[issued] = chunk_g[issued](bufs[bi], gsem[bi])
                issued += 1
            if j > 0 and seg_of[j] != seg_of[j - 1]:
                for i in range(max(0, j - NBUF), j):
                    wait_s(i)
                out, p = segs[seg_of[j - 1]]
                pltpu.sync_copy(accme, out.at[pl.ds(base + p * HPW, HPW)])
                if segs[seg_of[j]] is not None:
                    pltpu.sync_copy(z_h, accme)
            g[j].wait()
            s[j] = chunk_d[j](bufs[j % NBUF], ssem[j % NBUF])
        for i in range(max(0, n - NBUF), n):
            wait_s(i)

    return k(atab, ctab, ptab, aidx2d, cidx2d, pidx2d, dstidx, zeros)


def _mlp_pre(desc, W1d, b1):
    B, DD = desc.shape
    L1 = W1d.shape[1]
    BM = 1024

    def body(desc_ref, W_ref, b_ref, out_ref):
        x = desc_ref[...].astype(jnp.bfloat16)
        h = jnp.dot(x, W_ref[...], preferred_element_type=jnp.float32)
        out_ref[...] = (h + b_ref[...]).astype(jnp.bfloat16)

    return pl.pallas_call(
        body,
        grid=(B // BM,),
        in_specs=[
            pl.BlockSpec((BM, DD), lambda i: (i, 0)),
            pl.BlockSpec((DD, L1), lambda i: (0, 0)),
            pl.BlockSpec((1, L1), lambda i: (0, 0)),
        ],
        out_specs=pl.BlockSpec((BM, L1), lambda i: (i, 0)),
        out_shape=jax.ShapeDtypeStruct((B, L1), jnp.bfloat16),
    )(desc, W1d.astype(jnp.bfloat16), b1.reshape(1, -1))


def _mlp_post(h1, asum, prow, csum, aidx, cidx, W1e, W2, b2):
    B, L1 = h1.shape
    BM = 1024
    L = aidx.shape[1]
    E = W1e.shape[0]
    L2 = W2.shape[1]

    def body(h1_ref, asum_ref, p_ref, csum_ref, aidx_ref, cidx_ref,
             W1e_ref, W2_ref, b2_ref, out_ref):
        acnt = jnp.maximum(jnp.sum((aidx_ref[...] != 0).astype(jnp.float32),
                                   axis=1, keepdims=True), 1.0)
        ccnt = jnp.maximum(jnp.sum((cidx_ref[...] != 0).astype(jnp.float32),
                                   axis=1, keepdims=True), 1.0)
        a = asum_ref[...] / acnt
        c = csum_ref[...] / ccnt
        e = jnp.concatenate([a, p_ref[...], c], axis=1).astype(jnp.bfloat16)
        h = h1_ref[...].astype(jnp.float32) + jnp.dot(
            e, W1e_ref[...], preferred_element_type=jnp.float32)
        h = jnp.maximum(h, 0.0).astype(jnp.bfloat16)
        out_ref[...] = jnp.dot(h, W2_ref[...],
                               preferred_element_type=jnp.float32) + b2_ref[...]

    return pl.pallas_call(
        body,
        grid=(B // BM,),
        in_specs=[
            pl.BlockSpec((BM, L1), lambda i: (i, 0)),
            pl.BlockSpec((BM, D), lambda i: (i, 0)),
            pl.BlockSpec((BM, D), lambda i: (i, 0)),
            pl.BlockSpec((BM, D), lambda i: (i, 0)),
            pl.BlockSpec((BM, L), lambda i: (i, 0)),
            pl.BlockSpec((BM, L), lambda i: (i, 0)),
            pl.BlockSpec((E, L1), lambda i: (0, 0)),
            pl.BlockSpec((L1, L2), lambda i: (0, 0)),
            pl.BlockSpec((1, L2), lambda i: (0, 0)),
        ],
        out_specs=pl.BlockSpec((BM, L2), lambda i: (i, 0)),
        out_shape=jax.ShapeDtypeStruct((B, L2), jnp.float32),
    )(h1, asum, prow, csum, aidx, cidx, W1e.astype(jnp.bfloat16),
      W2.astype(jnp.bfloat16), b2.reshape(1, -1))


def kernel(description_embedding, authors, publishers, categories,
           authors_table, publishers_table, categories_table,
           W1, b1, W2, b2):
    B, L = authors.shape
    aidx32 = authors.astype(jnp.int32)
    cidx32 = categories.astype(jnp.int32)
    aidx2d = aidx32.reshape(-1, IDXW)
    cidx2d = cidx32.reshape(-1, IDXW)
    pidx2d = publishers.astype(jnp.int32).reshape(-1, IDXW)
    BPW = B // NW
    ROWS = BPW * L // IDXW
    HPW = BPW // 2
    pat = ((jnp.arange(BPW * L, dtype=jnp.int32) // L) % HPW).reshape(ROWS, IDXW)
    dstidx = pat[None, :, :] + (jnp.arange(NS, dtype=jnp.int32) * HPW)[:, None, None]
    zeros = jnp.zeros((HPW, D), jnp.float32)
    asum, prow, csum = _sc_gather(authors_table, categories_table,
                                  publishers_table, aidx2d, cidx2d, pidx2d,
                                  dstidx, zeros, B, L)
    DD = description_embedding.shape[1]
    h1 = _mlp_pre(description_embedding, W1[:DD], b1)
    return _mlp_post(h1, asum, prow, csum, aidx32, cidx32, W1[DD:], W2, b2)


# speedup vs baseline: 1.0779x; 1.0208x over previous
import functools

import jax
import jax.numpy as jnp
from jax import lax
from jax.experimental import pallas as pl
from jax.experimental.pallas import tpu as pltpu
from jax.experimental.pallas import tpu_sc as plsc

D = 128
NC, NS = 2, 16
NW = NC * NS
IDXW = 128


def _sc_gather(atab, ctab, ptab, aidx2d, cidx2d, pidx2d, dstidx, zeros, B, L):
    BPW = B // NW
    ROWS = BPW * L // IDXW
    PROWS = BPW // IDXW
    NPASS = 2
    HPW = BPW // NPASS
    ROWSP = ROWS // NPASS
    mesh = plsc.VectorSubcoreMesh(core_axis_name="c", subcore_axis_name="s")

    @functools.partial(
        pl.kernel,
        mesh=mesh,
        out_type=(
            jax.ShapeDtypeStruct((B, D), jnp.float32),
            jax.ShapeDtypeStruct((B, D), jnp.float32),
            jax.ShapeDtypeStruct((B, D), jnp.float32),
        ),
        scratch_types=[
            pltpu.VMEM((ROWS, IDXW), jnp.int32),
            pltpu.VMEM((ROWS, IDXW), jnp.int32),
            pltpu.VMEM((ROWS, IDXW), jnp.int32),
            pltpu.VMEM((PROWS, IDXW), jnp.int32),
            pltpu.VMEM((IDXW, D), jnp.float32),
            pltpu.VMEM((IDXW, D), jnp.float32),
            pltpu.VMEM((IDXW, D), jnp.float32),
            pltpu.VMEM((IDXW, D), jnp.float32),
            pltpu.VMEM_SHARED((NS * HPW, D), jnp.float32),
            pltpu.SemaphoreType.DMA,
            pltpu.SemaphoreType.DMA,
            pltpu.SemaphoreType.DMA,
            pltpu.SemaphoreType.DMA,
            pltpu.SemaphoreType.DMA,
            pltpu.SemaphoreType.DMA,
            pltpu.SemaphoreType.DMA,
            pltpu.SemaphoreType.DMA,
            pltpu.SemaphoreType.DMA,
        ],
    )
    def k(atab_h, ctab_h, ptab_h, aidx_h, cidx_h, pidx_h, dst_h, z_h,
          a_out, p_out, c_out, idx_va, idx_vc, dst_v, idx_vp,
          buf0, buf1, buf2, buf3,
          acc, g0, g1, g2, g3, s0, s1, s2, s3, msem):
        cix = lax.axis_index("c")
        six = lax.axis_index("s")
        wid = six * NC + cix
        base = wid * BPW
        bufs = (buf0, buf1, buf2, buf3)
        gsem = (g0, g1, g2, g3)
        ssem = (s0, s1, s2, s3)
        NBUF = 4
        accme = acc.at[pl.ds(six * HPW, HPW)]

        pre = [
            pltpu.async_copy(dst_h.at[six], dst_v, msem),
            pltpu.async_copy(aidx_h.at[pl.ds(wid * ROWS, ROWS)], idx_va, msem),
            pltpu.async_copy(cidx_h.at[pl.ds(wid * ROWS, ROWS)], idx_vc, msem),
            pltpu.async_copy(pidx_h.at[pl.ds(wid * PROWS, PROWS)], idx_vp, msem),
            pltpu.async_copy(z_h, accme, msem),
        ]
        for cp in pre:
            cp.wait()

        chunk_g = []
        chunk_d = []
        seg_of = []
        segs = []
        for p in range(NPASS):
            for tab_h, idx_v, out in ((atab_h, idx_va, a_out),
                                      (ctab_h, idx_vc, c_out)):
                for jj in range(ROWSP):
                    r = p * ROWSP + jj
                    chunk_g.append(
                        lambda b, sm, t=tab_h, iv=idx_v, r=r:
                            pltpu.async_copy(t.at[iv.at[r]], b, sm))
                    chunk_d.append(
                        lambda b, sm, r=r: pltpu.async_copy(
                            b, acc.at[dst_v.at[r]], sm, add=True))
                    seg_of.append(len(segs))
                segs.append((out, p))
        for jj in range(PROWS):
            chunk_g.append(
                lambda b, sm, r=jj:
                    pltpu.async_copy(ptab_h.at[idx_vp.at[r]], b, sm))
            chunk_d.append(
                lambda b, sm, r=jj: pltpu.async_copy(
                    b, p_out.at[pl.ds(base + r * IDXW, IDXW)], sm))
            seg_of.append(len(segs))
        segs.append(None)

        n = len(chunk_g)
        g = [None] * n
        s = [None] * n
        s_waited = [False] * n

        def wait_s(i):
            if not s_waited[i]:
                s[i].wait()
                s_waited[i] = True

        issued = 0
        for j in range(n):
            while issued < n and issued < j + NBUF:
                bi = issued % NBUF
                if issued >= NBUF:
                    wait_s(issued - NBUF)
                g[issued] = chunk_g[issued](bufs[bi], gsem[bi])
                issued += 1
            if j > 0 and seg_of[j] != seg_of[j - 1]:
                for i in range(max(0, j - NBUF), j):
                    wait_s(i)
                out, p = segs[seg_of[j - 1]]
                pltpu.sync_copy(accme, out.at[pl.ds(base + p * HPW, HPW)])
                if segs[seg_of[j]] is not None:
                    pltpu.sync_copy(z_h, accme)
            g[j].wait()
            s[j] = chunk_d[j](bufs[j % NBUF], ssem[j % NBUF])
        for i in range(max(0, n - NBUF), n):
            wait_s(i)

    return k(atab, ctab, ptab, aidx2d, cidx2d, pidx2d, dstidx, zeros)


def _mlp_pre(desc, W1d, b1):
    B, DD = desc.shape
    L1 = W1d.shape[1]
    BM = 2048

    def body(desc_ref, W_ref, b_ref, out_ref):
        x = desc_ref[...].astype(jnp.bfloat16)
        h = jnp.dot(x, W_ref[...], preferred_element_type=jnp.float32)
        out_ref[...] = (h + b_ref[...]).astype(jnp.bfloat16)

    return pl.pallas_call(
        body,
        grid=(B // BM,),
        in_specs=[
            pl.BlockSpec((BM, DD), lambda i: (i, 0)),
            pl.BlockSpec((DD, L1), lambda i: (0, 0)),
            pl.BlockSpec((1, L1), lambda i: (0, 0)),
        ],
        out_specs=pl.BlockSpec((BM, L1), lambda i: (i, 0)),
        out_shape=jax.ShapeDtypeStruct((B, L1), jnp.bfloat16),
    )(desc, W1d.astype(jnp.bfloat16), b1.reshape(1, -1))


def _mlp_post(h1, asum, prow, csum, aidx, cidx, W1e, W2, b2):
    B, L1 = h1.shape
    BM = 2048
    L = aidx.shape[1]
    E = W1e.shape[0]
    L2 = W2.shape[1]

    def body(h1_ref, asum_ref, p_ref, csum_ref, aidx_ref, cidx_ref,
             W1e_ref, W2_ref, b2_ref, out_ref):
        acnt = jnp.maximum(jnp.sum((aidx_ref[...] != 0).astype(jnp.float32),
                                   axis=1, keepdims=True), 1.0)
        ccnt = jnp.maximum(jnp.sum((cidx_ref[...] != 0).astype(jnp.float32),
                                   axis=1, keepdims=True), 1.0)
        a = asum_ref[...] / acnt
        c = csum_ref[...] / ccnt
        e = jnp.concatenate([a, p_ref[...], c], axis=1).astype(jnp.bfloat16)
        h = h1_ref[...].astype(jnp.float32) + jnp.dot(
            e, W1e_ref[...], preferred_element_type=jnp.float32)
        h = jnp.maximum(h, 0.0).astype(jnp.bfloat16)
        out_ref[...] = jnp.dot(h, W2_ref[...],
                               preferred_element_type=jnp.float32) + b2_ref[...]

    return pl.pallas_call(
        body,
        grid=(B // BM,),
        in_specs=[
            pl.BlockSpec((BM, L1), lambda i: (i, 0)),
            pl.BlockSpec((BM, D), lambda i: (i, 0)),
            pl.BlockSpec((BM, D), lambda i: (i, 0)),
            pl.BlockSpec((BM, D), lambda i: (i, 0)),
            pl.BlockSpec((BM, L), lambda i: (i, 0)),
            pl.BlockSpec((BM, L), lambda i: (i, 0)),
            pl.BlockSpec((E, L1), lambda i: (0, 0)),
            pl.BlockSpec((L1, L2), lambda i: (0, 0)),
            pl.BlockSpec((1, L2), lambda i: (0, 0)),
        ],
        out_specs=pl.BlockSpec((BM, L2), lambda i: (i, 0)),
        out_shape=jax.ShapeDtypeStruct((B, L2), jnp.float32),
    )(h1, asum, prow, csum, aidx, cidx, W1e.astype(jnp.bfloat16),
      W2.astype(jnp.bfloat16), b2.reshape(1, -1))


def kernel(description_embedding, authors, publishers, categories,
           authors_table, publishers_table, categories_table,
           W1, b1, W2, b2):
    B, L = authors.shape
    aidx32 = authors.astype(jnp.int32)
    cidx32 = categories.astype(jnp.int32)
    aidx2d = aidx32.reshape(-1, IDXW)
    cidx2d = cidx32.reshape(-1, IDXW)
    pidx2d = publishers.astype(jnp.int32).reshape(-1, IDXW)
    BPW = B // NW
    ROWS = BPW * L // IDXW
    HPW = BPW // 2
    pat = ((jnp.arange(BPW * L, dtype=jnp.int32) // L) % HPW).reshape(ROWS, IDXW)
    dstidx = pat[None, :, :] + (jnp.arange(NS, dtype=jnp.int32) * HPW)[:, None, None]
    zeros = jnp.zeros((HPW, D), jnp.float32)
    asum, prow, csum = _sc_gather(authors_table, categories_table,
                                  publishers_table, aidx2d, cidx2d, pidx2d,
                                  dstidx, zeros, B, L)
    DD = description_embedding.shape[1]
    h1 = _mlp_pre(description_embedding, W1[:DD], b1)
    return _mlp_post(h1, asum, prow, csum, aidx32, cidx32, W1[DD:], W2, b2)
